# baseline (speedup 1.0000x reference)
import jax
import jax.numpy as jnp
from jax import lax
from jax.experimental import pallas as pl
from jax.experimental.pallas import tpu as pltpu
from jax.experimental.pallas import tpu_sc as plsc

NBITS = 256
HIDDEN = 256
SPATIAL = 32
B = 128
C = 128

NB = 2
GRID = B // NB

NC = 2
NS = 16
NW = NC * NS
BPW = B // NW
LANES = 16


def _sc_lookup_body(msg_hbm, w_hbm, out_hbm, msg_v, idx_v, rows_v, acc_v, sem):
    wid = lax.axis_index("s") * NC + lax.axis_index("c")
    lane = lax.iota(jnp.int32, LANES)
    nh = HIDDEN // LANES
    for j in range(BPW):
        b = wid * BPW + j
        pltpu.sync_copy(msg_hbm.at[b], msg_v)
        for t in range(NBITS // LANES):
            idx_v[pl.ds(t * LANES, LANES)] = (
                2 * (t * LANES + lane) + msg_v[pl.ds(t * LANES, LANES)])
        pltpu.async_copy(w_hbm.at[idx_v], rows_v, sem).wait()

        def _row(r, accs):
            return tuple(
                accs[t] + rows_v[r, pl.ds(t * LANES, LANES)]
                for t in range(nh))

        accs = tuple(jnp.zeros((LANES,), jnp.float32) for _ in range(nh))
        accs = lax.fori_loop(0, NBITS, _row, accs)
        for t in range(nh):
            acc_v[pl.ds(t * LANES, LANES)] = accs[t]
        pltpu.sync_copy(acc_v, out_hbm.at[b])


def _sc_lookup(msg, w):
    mesh = plsc.VectorSubcoreMesh(core_axis_name="c", subcore_axis_name="s",
                                  num_cores=NC)
    return pl.kernel(
        _sc_lookup_body,
        out_type=jax.ShapeDtypeStruct((B, HIDDEN), jnp.float32),
        mesh=mesh,
        scratch_types=[
            pltpu.VMEM((NBITS,), jnp.int32),
            pltpu.VMEM((NBITS,), jnp.int32),
            pltpu.VMEM((NBITS, HIDDEN), jnp.float32),
            pltpu.VMEM((HIDDEN,), jnp.float32),
            pltpu.SemaphoreType.DMA,
        ],
    )(msg, w)


def _bcast_body(aux_ref, lat_ref, out_ref):
    out_ref[:, :C] = lat_ref[...]
    for i in range(NB):
        col = aux_ref[i, 0].reshape(HIDDEN, 1)
        t2 = jnp.broadcast_to(col, (HIDDEN, SPATIAL))
        for y in range(SPATIAL):
            out_ref[i, C:, y, :] = t2


def kernel(latents, msg, W):
    aux = _sc_lookup(msg.astype(jnp.int32), W)
    aux3 = aux.reshape(B, 1, HIDDEN)
    out = pl.pallas_call(
        _bcast_body,
        grid=(GRID,),
        in_specs=[
            pl.BlockSpec((NB, 1, HIDDEN), lambda g: (g, 0, 0)),
            pl.BlockSpec((NB, C, SPATIAL, SPATIAL), lambda g: (g, 0, 0, 0)),
        ],
        out_specs=pl.BlockSpec((NB, C + HIDDEN, SPATIAL, SPATIAL),
                               lambda g: (g, 0, 0, 0)),
        out_shape=jax.ShapeDtypeStruct((B, C + HIDDEN, SPATIAL, SPATIAL),
                                       jnp.float32),
    )(aux3, latents)
    return out


# speedup vs baseline: 3.2331x; 3.2331x over previous
import jax
import jax.numpy as jnp
from jax import lax
from jax.experimental import pallas as pl
from jax.experimental.pallas import tpu as pltpu
from jax.experimental.pallas import tpu_sc as plsc

NBITS = 256
HIDDEN = 256
SPATIAL = 32
B = 128
C = 128
HW = SPATIAL * SPATIAL

NB = 8
GRID = B // NB

NC = 2
NS = 16
NW = NC * NS
BPW = B // NW
LANES = 16


def _sc_lookup_body(msg_hbm, w_hbm, out_hbm, msg_v, idx_v, rows_v, acc_v, sem):
    wid = lax.axis_index("s") * NC + lax.axis_index("c")
    lane = lax.iota(jnp.int32, LANES)
    nh = HIDDEN // LANES
    for j in range(BPW):
        b = wid * BPW + j
        pltpu.sync_copy(msg_hbm.at[b], msg_v)
        for t in range(NBITS // LANES):
            idx_v[pl.ds(t * LANES, LANES)] = (
                2 * (t * LANES + lane) + msg_v[pl.ds(t * LANES, LANES)])
        pltpu.async_copy(w_hbm.at[idx_v], rows_v, sem).wait()

        def _row(r, accs):
            return tuple(
                accs[t] + rows_v[r, pl.ds(t * LANES, LANES)]
                for t in range(nh))

        accs = tuple(jnp.zeros((LANES,), jnp.float32) for _ in range(nh))
        accs = lax.fori_loop(0, NBITS, _row, accs)
        for t in range(nh):
            acc_v[pl.ds(t * LANES, LANES)] = accs[t]
        pltpu.sync_copy(acc_v, out_hbm.at[b])


def _sc_lookup(msg, w):
    mesh = plsc.VectorSubcoreMesh(core_axis_name="c", subcore_axis_name="s",
                                  num_cores=NC)
    return pl.kernel(
        _sc_lookup_body,
        out_type=jax.ShapeDtypeStruct((B, HIDDEN), jnp.float32),
        mesh=mesh,
        scratch_types=[
            pltpu.VMEM((NBITS,), jnp.int32),
            pltpu.VMEM((NBITS,), jnp.int32),
            pltpu.VMEM((NBITS, HIDDEN), jnp.float32),
            pltpu.VMEM((HIDDEN,), jnp.float32),
            pltpu.SemaphoreType.DMA,
        ],
    )(msg, w)


def _bcast_body(aux_ref, lat_ref, out_ref):
    out_ref[:, :C, :] = lat_ref[...]
    out_ref[:, C:, :] = jnp.broadcast_to(aux_ref[...], (NB, HIDDEN, HW))


def kernel(latents, msg, W):
    lat3 = latents.reshape(B, C, HW)
    aux = _sc_lookup(msg.astype(jnp.int32), W)
    aux3 = aux.reshape(B, HIDDEN, 1)
    out = pl.pallas_call(
        _bcast_body,
        grid=(GRID,),
        in_specs=[
            pl.BlockSpec((NB, HIDDEN, 1), lambda g: (g, 0, 0)),
            pl.BlockSpec((NB, C, HW), lambda g: (g, 0, 0)),
        ],
        out_specs=pl.BlockSpec((NB, C + HIDDEN, HW), lambda g: (g, 0, 0)),
        out_shape=jax.ShapeDtypeStruct((B, C + HIDDEN, HW), jnp.float32),
    )(aux3, lat3)
    return out.reshape(B, C + HIDDEN, SPATIAL, SPATIAL)
